# baseline (speedup 1.0000x reference)
import functools

import jax
import jax.numpy as jnp
from jax import lax
from jax.experimental import pallas as pl
from jax.experimental.pallas import tpu as pltpu
from jax.experimental.pallas import tpu_sc as plsc

N = 10000
E = 320000
D = 128
ED = 16

NC = 2
NS = 16
NW = NC * NS
CHUNK = E // NW
BE = 80
NBLK = CHUNK // BE
NBLK_A = 63
NBLK_B = NBLK - NBLK_A
E_A = NW * NBLK_A * BE
E_B = E - E_A
ROWS = N // NS


def _mesh():
    return plsc.VectorSubcoreMesh(core_axis_name="c", subcore_axis_name="s",
                                  num_cores=NC, num_subcores=NS)


def _sc_gather_body(nblk, p_hbm, q_hbm, xpad_hbm, row2_hbm, col2_hbm,
                    g_hbm, df_hbm,
                    idxr, idxc,
                    pb0, qb0, xb0, yb0, sem0,
                    pb1, qb1, xb1, yb1, sem1):
    c = lax.axis_index("c")
    s = lax.axis_index("s")
    wid = s * NC + c
    chunk = nblk * BE
    pltpu.sync_copy(row2_hbm.at[wid], idxr)
    pltpu.sync_copy(col2_hbm.at[wid], idxc)
    bufs = ((pb0, qb0, xb0, yb0, sem0), (pb1, qb1, xb1, yb1, sem1))

    def fire(j, bu):
        pb, qb, xb, yb, sem = bu
        pltpu.async_copy(p_hbm.at[idxr.at[j]], pb, sem)
        pltpu.async_copy(q_hbm.at[idxc.at[j]], qb, sem)
        pltpu.async_copy(xpad_hbm.at[idxr.at[j]], xb, sem)
        pltpu.async_copy(xpad_hbm.at[idxc.at[j]], yb, sem)

    def drain(j, bu):
        pb, qb, xb, yb, sem = bu
        pltpu.make_async_copy(p_hbm.at[idxr.at[j]], pb, sem).wait()
        pltpu.make_async_copy(q_hbm.at[idxc.at[j]], qb, sem).wait()
        pltpu.make_async_copy(xpad_hbm.at[idxr.at[j]], xb, sem).wait()
        pltpu.make_async_copy(xpad_hbm.at[idxc.at[j]], yb, sem).wait()

        @plsc.parallel_loop(0, BE, unroll=4)
        def _(e):
            for k in range(D // 32):
                sl = pl.ds(k * 32, 32)
                pb[e, sl] = pb[e, sl] + qb[e, sl]
            xb[e, :] = xb[e, :] - yb[e, :]

        base = wid * chunk + j * BE
        pltpu.sync_copy(pb, g_hbm.at[pl.ds(base, BE)])
        pltpu.sync_copy(xb, df_hbm.at[pl.ds(base, BE)])

    fire(0, bufs[0])

    def body(i, carry):
        for b in range(2):
            j = i * 2 + b

            @pl.when(j + 1 < nblk)
            def _():
                fire(j + 1, bufs[1 - b])

            @pl.when(j < nblk)
            def _():
                drain(j, bufs[b])
        return carry

    lax.fori_loop(0, (nblk + 1) // 2, body, 0)


@functools.lru_cache(maxsize=None)
def _sc_gather(nblk):
    esz = NW * nblk * BE
    return pl.kernel(
        functools.partial(_sc_gather_body, nblk),
        out_type=[
            jax.ShapeDtypeStruct((esz, D), jnp.bfloat16),
            jax.ShapeDtypeStruct((esz, ED), jnp.float32),
        ],
        mesh=_mesh(),
        scratch_types=[
            pltpu.VMEM((nblk, BE), jnp.int32),
            pltpu.VMEM((nblk, BE), jnp.int32),
            pltpu.VMEM((BE, D), jnp.bfloat16),
            pltpu.VMEM((BE, D), jnp.bfloat16),
            pltpu.VMEM((BE, ED), jnp.float32),
            pltpu.VMEM((BE, ED), jnp.float32),
            pltpu.SemaphoreType.DMA,
            pltpu.VMEM((BE, D), jnp.bfloat16),
            pltpu.VMEM((BE, D), jnp.bfloat16),
            pltpu.VMEM((BE, ED), jnp.float32),
            pltpu.VMEM((BE, ED), jnp.float32),
            pltpu.SemaphoreType.DMA,
        ],
        compiler_params=pltpu.CompilerParams(use_tc_tiling_on_sc=False),
    )


def _sc_scatter_body(nblk, full_init, msg_hbm, wd_hbm, col2_hbm, i128_hbm, i16_hbm,
                     aggp_hbm, cdp_hbm,
                     idxc, mb0, wb0, sem0, mb1, wb1, sem1, sh128, sh16):
    c = lax.axis_index("c")
    s = lax.axis_index("s")
    wid = s * NC + c
    chunk = nblk * BE
    r0 = s * ROWS
    i0 = c * N + r0 if full_init else r0
    pltpu.sync_copy(col2_hbm.at[wid], idxc)
    pltpu.sync_copy(i128_hbm.at[pl.ds(i0, ROWS)], sh128.at[pl.ds(r0, ROWS)])
    pltpu.sync_copy(i16_hbm.at[pl.ds(i0, ROWS)], sh16.at[pl.ds(r0, ROWS)])
    plsc.subcore_barrier()
    bufs = ((mb0, wb0, sem0), (mb1, wb1, sem1))

    def fire(j, bu):
        mb, wb, sem = bu
        base = wid * chunk + j * BE
        pltpu.async_copy(msg_hbm.at[pl.ds(base, BE)], mb, sem)
        pltpu.async_copy(wd_hbm.at[pl.ds(base, BE)], wb, sem)

    def drain(j, bu):
        mb, wb, sem = bu
        base = wid * chunk + j * BE
        pltpu.make_async_copy(msg_hbm.at[pl.ds(base, BE)], mb, sem).wait()
        pltpu.make_async_copy(wd_hbm.at[pl.ds(base, BE)], wb, sem).wait()
        pltpu.sync_copy(mb, sh128.at[idxc.at[j]], add=True)
        pltpu.sync_copy(wb, sh16.at[idxc.at[j]], add=True)

    fire(0, bufs[0])

    def body(i, carry):
        for b in range(2):
            j = i * 2 + b

            @pl.when(j + 1 < nblk)
            def _():
                fire(j + 1, bufs[1 - b])

            @pl.when(j < nblk)
            def _():
                drain(j, bufs[b])
        return carry

    lax.fori_loop(0, (nblk + 1) // 2, body, 0)
    plsc.subcore_barrier()
    off = c * N + r0
    pltpu.sync_copy(sh128.at[pl.ds(r0, ROWS)], aggp_hbm.at[pl.ds(off, ROWS)])
    pltpu.sync_copy(sh16.at[pl.ds(r0, ROWS)], cdp_hbm.at[pl.ds(off, ROWS)])


@functools.lru_cache(maxsize=None)
def _sc_scatter(nblk, full_init):
    return pl.kernel(
        functools.partial(_sc_scatter_body, nblk, full_init),
        out_type=[
            jax.ShapeDtypeStruct((NC * N, D), jnp.float32),
            jax.ShapeDtypeStruct((NC * N, ED), jnp.float32),
        ],
        mesh=_mesh(),
        scratch_types=[
            pltpu.VMEM((nblk, BE), jnp.int32),
            pltpu.VMEM((BE, D), jnp.float32),
            pltpu.VMEM((BE, ED), jnp.float32),
            pltpu.SemaphoreType.DMA,
            pltpu.VMEM((BE, D), jnp.float32),
            pltpu.VMEM((BE, ED), jnp.float32),
            pltpu.SemaphoreType.DMA,
            pltpu.VMEM_SHARED((N, D), jnp.float32),
            pltpu.VMEM_SHARED((N, ED), jnp.float32),
        ],
        compiler_params=pltpu.CompilerParams(use_tc_tiling_on_sc=False),
    )


_NB_PRE = 10
_BN = N // _NB_PRE


def _tc_pre_body(h_ref, w1a, w1b, wn1a, bn1, p_ref, q_ref, r_ref):
    h = h_ref[:]
    p_ref[:] = jnp.dot(h, w1a[:], preferred_element_type=jnp.float32
                       ).astype(jnp.bfloat16)
    q_ref[:] = jnp.dot(h, w1b[:], preferred_element_type=jnp.float32
                       ).astype(jnp.bfloat16)
    r_ref[:] = jnp.dot(h, wn1a[:], preferred_element_type=jnp.float32) + bn1[:]


def _tc_pre(h, w1a, w1b, wn1a, bn1):
    full = lambda shp: pl.BlockSpec(shp, lambda i: (0, 0))
    blk = pl.BlockSpec((_BN, D), lambda i: (i, 0))
    return pl.pallas_call(
        _tc_pre_body,
        grid=(_NB_PRE,),
        in_specs=[blk, full((D, D)), full((D, D)), full((D, D)), full((1, D))],
        out_specs=[blk, blk, blk],
        out_shape=[jax.ShapeDtypeStruct((N, D), jnp.bfloat16),
                   jax.ShapeDtypeStruct((N, D), jnp.bfloat16),
                   jax.ShapeDtypeStruct((N, D), jnp.float32)],
    )(h, w1a, w1b, wn1a, bn1)


def _ln(t, g, b):
    mu = jnp.mean(t, axis=-1, keepdims=True)
    var = jnp.mean((t - mu) ** 2, axis=-1, keepdims=True)
    return (t - mu) / jnp.sqrt(var + 1e-5) * g + b


def _silu(t):
    return t * jax.nn.sigmoid(t)


def _tc_edge_body(g_ref, df_ref, ea_ref,
                  w1d, w1e, b1, gm, btm, w2e, b2e,
                  wc1, bc1, wc2p, bc2,
                  msg_ref, wd_ref):
    diff = df_ref[:]
    dsq = jnp.sum(diff * diff, axis=-1, keepdims=True)
    t = (g_ref[:].astype(jnp.float32)
         + dsq * w1d[:]
         + jnp.dot(ea_ref[:], w1e[:], preferred_element_type=jnp.float32)
         + b1[:])
    t = _silu(_ln(t, gm[:], btm[:]))
    m2 = jnp.dot(t, w2e[:], preferred_element_type=jnp.float32) + b2e[:]
    att = jax.nn.sigmoid(m2[:, D:D + 1])
    msg = m2[:, :D] * att
    msg_ref[:] = msg
    c1 = _silu(jnp.dot(msg, wc1[:], preferred_element_type=jnp.float32) + bc1[:])
    cwf = jnp.dot(c1, wc2p[:], preferred_element_type=jnp.float32)
    cw = jnp.clip(cwf[:, 0:1] + bc2[:], -100.0, 100.0)
    lanes = lax.broadcasted_iota(jnp.int32, (1, ED), 1)
    wd_ref[:] = jnp.where(lanes < 3, diff * cw, 0.0)


def _tc_edge(blk_rows, blk_off, g, df, ea,
             w1d, w1e, b1, gm, btm, w2e, b2e, wc1, bc1, wc2p, bc2):
    esz = g.shape[0]
    full = lambda shp: pl.BlockSpec(shp, lambda i: (0, 0))
    blkD = pl.BlockSpec((blk_rows, D), lambda i: (i, 0))
    blkE = pl.BlockSpec((blk_rows, ED), lambda i: (i, 0))
    blkEoff = pl.BlockSpec((blk_rows, ED), lambda i: (i + blk_off, 0))
    return pl.pallas_call(
        _tc_edge_body,
        grid=(esz // blk_rows,),
        in_specs=[blkD, blkE, blkEoff,
                  full((1, D)), full((ED, D)), full((1, D)), full((1, D)),
                  full((1, D)), full((D, 2 * D)), full((1, 2 * D)),
                  full((D, D)), full((1, D)),
                  full((D, D)), full((1, 1))],
        out_specs=[blkD, blkE],
        out_shape=[jax.ShapeDtypeStruct((esz, D), jnp.float32),
                   jax.ShapeDtypeStruct((esz, ED), jnp.float32)],
    )(g, df, ea, w1d, w1e, b1, gm, btm, w2e, b2e, wc1, bc1, wc2p, bc2)


def _tc_node_body(h_ref, r_ref, a0_ref, a1_ref, c0_ref, c1_ref, xp_ref,
                  wn1b, gm, btm, wn2, b2n,
                  hout, xout):
    t = r_ref[:] + jnp.dot(a0_ref[:] + a1_ref[:], wn1b[:],
                           preferred_element_type=jnp.float32)
    t = _silu(_ln(t, gm[:], btm[:]))
    hout[:] = h_ref[:] + jnp.dot(t, wn2[:], preferred_element_type=jnp.float32) + b2n[:]
    xout[:] = xp_ref[:] + c0_ref[:] + c1_ref[:]


def _tc_node(h, r, aggp, cdp, xp, wn1b, gm, btm, wn2, b2n):
    full = lambda shp: pl.BlockSpec(shp, lambda i: (0, 0))
    blkD = pl.BlockSpec((_BN, D), lambda i: (i, 0))
    blkE = pl.BlockSpec((_BN, ED), lambda i: (i, 0))
    blkD1 = pl.BlockSpec((_BN, D), lambda i: (i + _NB_PRE, 0))
    blkE1 = pl.BlockSpec((_BN, ED), lambda i: (i + _NB_PRE, 0))
    return pl.pallas_call(
        _tc_node_body,
        grid=(_NB_PRE,),
        in_specs=[blkD, blkD, blkD, blkD1, blkE, blkE1, blkE,
                  full((D, D)), full((1, D)), full((1, D)), full((D, D)),
                  full((1, D))],
        out_specs=[blkD, blkE],
        out_shape=[jax.ShapeDtypeStruct((N, D), jnp.float32),
                   jax.ShapeDtypeStruct((N, ED), jnp.float32)],
    )(h, r, aggp, aggp, cdp, cdp, xp, wn1b, gm, btm, wn2, b2n)


def kernel(h, x, edge_index, edge_attr,
           W_msg1, b_msg1, g_msg1, be_msg1, W_msg2, b_msg2,
           W_att, b_att,
           W_node1, b_node1, g_node1, be_node1, W_node2, b_node2,
           W_coord1, b_coord1, W_coord2, b_coord2):
    row = edge_index[0]
    col = edge_index[1]
    w1a = W_msg1[0:D]
    w1b = W_msg1[D:2 * D]
    w1d = W_msg1[2 * D:2 * D + 1]
    w1e = W_msg1[2 * D + 1:]
    wn1a = W_node1[0:D]
    wn1b = W_node1[D:]
    xpad = jnp.pad(x, ((0, 0), (0, ED - 3)))

    row2 = lambda v: v.reshape(1, -1)

    w2e = jnp.concatenate(
        [W_msg2, W_msg2 @ W_att, jnp.zeros((D, D - 1), jnp.float32)], axis=1)
    b2e = jnp.concatenate(
        [b_msg2, b_msg2 @ W_att + b_att, jnp.zeros((D - 1,), jnp.float32)]
    ).reshape(1, 2 * D)
    wc2p = jnp.concatenate(
        [W_coord2, jnp.zeros((D, D - 1), jnp.float32)], axis=1)

    rowA = row[:E_A].reshape(NW, NBLK_A, BE)
    rowB = row[E_A:].reshape(NW, NBLK_B, BE)
    colA = col[:E_A].reshape(NW, NBLK_A, BE)
    colB = col[E_A:].reshape(NW, NBLK_B, BE)
    assert E_A % 2560 == 0 and E_B % 2560 == 0

    edge_w = (w1d, w1e, row2(b_msg1), row2(g_msg1), row2(be_msg1),
              w2e, b2e, W_coord1, row2(b_coord1), wc2p,
              b_coord2.reshape(1, 1))

    p, q, r = _tc_pre(h, w1a, w1b, wn1a, row2(b_node1))
    gA, dfA = _sc_gather(NBLK_A)(p, q, xpad, rowA, colA)
    gB, dfB = _sc_gather(NBLK_B)(p, q, xpad, rowB, colB)
    mA, wdA = _tc_edge(2560, 0, gA, dfA, edge_attr, *edge_w)
    z128 = jnp.zeros((N, D), jnp.float32)
    z16 = jnp.zeros((N, ED), jnp.float32)
    aggA, cdA = _sc_scatter(NBLK_A, False)(mA, wdA, colA, z128, z16)
    mB, wdB = _tc_edge(2560, E_A // 2560, gB, dfB, edge_attr, *edge_w)
    aggp, cdp = _sc_scatter(NBLK_B, True)(mB, wdB, colB, aggA, cdA)
    h_new, xnp = _tc_node(
        h, r, aggp, cdp, xpad,
        wn1b, row2(g_node1), row2(be_node1), W_node2, row2(b_node2))
    return (h_new, xnp[:, :3])


# speedup vs baseline: 1.5288x; 1.5288x over previous
import functools

import jax
import jax.numpy as jnp
from jax import lax
from jax.experimental import pallas as pl
from jax.experimental.pallas import tpu as pltpu
from jax.experimental.pallas import tpu_sc as plsc

N = 10000
E = 320000
D = 128
ED = 16

NC = 2
NS = 16
NW = NC * NS
CHUNK = E // NW
BE = 80
NBLK = CHUNK // BE
NBLK_A = 63
NBLK_B = NBLK - NBLK_A
E_A = NW * NBLK_A * BE
E_B = E - E_A
ROWS = N // NS


def _mesh():
    return plsc.VectorSubcoreMesh(core_axis_name="c", subcore_axis_name="s",
                                  num_cores=NC, num_subcores=NS)


def _vbcast(vec, idx):
    dn = lax.GatherDimensionNumbers(
        offset_dims=(), collapsed_slice_dims=(0,), start_index_map=(0,))
    return lax.gather(vec, idx[:, None], dn, (1,),
                      mode=lax.GatherScatterMode.PROMISE_IN_BOUNDS)


def _sc_gather_body(nblk, p_hbm, q_hbm, xpad_hbm, row2_hbm, col2_hbm, w1d_hbm,
                    g_hbm,
                    idxr, idxc, w1dv,
                    pb0, qb0, xb0, yb0, sem0,
                    pb1, qb1, xb1, yb1, sem1):
    c = lax.axis_index("c")
    s = lax.axis_index("s")
    wid = s * NC + c
    chunk = nblk * BE
    pltpu.sync_copy(row2_hbm.at[wid], idxr)
    pltpu.sync_copy(col2_hbm.at[wid], idxc)
    pltpu.sync_copy(w1d_hbm, w1dv)
    bufs = ((pb0, qb0, xb0, yb0, sem0), (pb1, qb1, xb1, yb1, sem1))
    z16 = jnp.zeros((16,), jnp.int32)
    o16 = jnp.ones((16,), jnp.int32)
    t16 = jnp.full((16,), 2, jnp.int32)

    def fire(j, bu):
        pb, qb, xb, yb, sem = bu
        pltpu.async_copy(p_hbm.at[idxr.at[j]], pb, sem)
        pltpu.async_copy(q_hbm.at[idxc.at[j]], qb, sem)
        pltpu.async_copy(xpad_hbm.at[idxr.at[j]], xb, sem)
        pltpu.async_copy(xpad_hbm.at[idxc.at[j]], yb, sem)

    def drain(j, bu):
        pb, qb, xb, yb, sem = bu
        pltpu.make_async_copy(p_hbm.at[idxr.at[j]], pb, sem).wait()
        pltpu.make_async_copy(q_hbm.at[idxc.at[j]], qb, sem).wait()
        pltpu.make_async_copy(xpad_hbm.at[idxr.at[j]], xb, sem).wait()
        pltpu.make_async_copy(xpad_hbm.at[idxc.at[j]], yb, sem).wait()
        w1s = [w1dv[0, pl.ds(k * 16, 16)] for k in range(D // 16)]

        @plsc.parallel_loop(0, BE, unroll=2)
        def _(e):
            d = xb[e, :] - yb[e, :]
            sq = d * d
            s2v = _vbcast(sq, z16) + _vbcast(sq, o16) + _vbcast(sq, t16)
            for k in range(D // 16):
                sl = pl.ds(k * 16, 16)
                pb[e, sl] = pb[e, sl] + qb[e, sl] + s2v * w1s[k]

        base = wid * chunk + j * BE
        pltpu.sync_copy(pb, g_hbm.at[pl.ds(base, BE)])

    fire(0, bufs[0])

    def body(i, carry):
        for b in range(2):
            j = i * 2 + b

            @pl.when(j + 1 < nblk)
            def _():
                fire(j + 1, bufs[1 - b])

            @pl.when(j < nblk)
            def _():
                drain(j, bufs[b])
        return carry

    lax.fori_loop(0, (nblk + 1) // 2, body, 0)


@functools.lru_cache(maxsize=None)
def _sc_gather(nblk):
    esz = NW * nblk * BE
    return pl.kernel(
        functools.partial(_sc_gather_body, nblk),
        out_type=[
            jax.ShapeDtypeStruct((esz, D), jnp.float32),
        ],
        mesh=_mesh(),
        scratch_types=[
            pltpu.VMEM((nblk, BE), jnp.int32),
            pltpu.VMEM((nblk, BE), jnp.int32),
            pltpu.VMEM((1, D), jnp.float32),
            pltpu.VMEM((BE, D), jnp.float32),
            pltpu.VMEM((BE, D), jnp.float32),
            pltpu.VMEM((BE, ED), jnp.float32),
            pltpu.VMEM((BE, ED), jnp.float32),
            pltpu.SemaphoreType.DMA,
            pltpu.VMEM((BE, D), jnp.float32),
            pltpu.VMEM((BE, D), jnp.float32),
            pltpu.VMEM((BE, ED), jnp.float32),
            pltpu.VMEM((BE, ED), jnp.float32),
            pltpu.SemaphoreType.DMA,
        ],
        compiler_params=pltpu.CompilerParams(use_tc_tiling_on_sc=False),
    )


def _sc_scatter_body(nblk, full_init, msg_hbm, cw_hbm, xpad_hbm,
                     row2_hbm, col2_hbm, i128_hbm, i16_hbm,
                     aggp_hbm, cdp_hbm,
                     idxr, idxc,
                     mb0, wb0, xb0, yb0, sem0,
                     mb1, wb1, xb1, yb1, sem1, sh128, sh16):
    c = lax.axis_index("c")
    s = lax.axis_index("s")
    wid = s * NC + c
    chunk = nblk * BE
    r0 = s * ROWS
    i0 = c * N + r0 if full_init else r0
    pltpu.sync_copy(row2_hbm.at[wid], idxr)
    pltpu.sync_copy(col2_hbm.at[wid], idxc)
    pltpu.sync_copy(i128_hbm.at[pl.ds(i0, ROWS)], sh128.at[pl.ds(r0, ROWS)])
    pltpu.sync_copy(i16_hbm.at[pl.ds(i0, ROWS)], sh16.at[pl.ds(r0, ROWS)])
    plsc.subcore_barrier()
    bufs = ((mb0, wb0, xb0, yb0, sem0), (mb1, wb1, xb1, yb1, sem1))
    z16 = jnp.zeros((16,), jnp.int32)

    def fire(j, bu):
        mb, wb, xb, yb, sem = bu
        base = wid * chunk + j * BE
        pltpu.async_copy(msg_hbm.at[pl.ds(base, BE)], mb, sem)
        pltpu.async_copy(cw_hbm.at[pl.ds(base, BE)], wb, sem)
        pltpu.async_copy(xpad_hbm.at[idxr.at[j]], xb, sem)
        pltpu.async_copy(xpad_hbm.at[idxc.at[j]], yb, sem)

    def drain(j, bu):
        mb, wb, xb, yb, sem = bu
        base = wid * chunk + j * BE
        pltpu.make_async_copy(msg_hbm.at[pl.ds(base, BE)], mb, sem).wait()
        pltpu.make_async_copy(cw_hbm.at[pl.ds(base, BE)], wb, sem).wait()
        pltpu.make_async_copy(xpad_hbm.at[idxr.at[j]], xb, sem).wait()
        pltpu.make_async_copy(xpad_hbm.at[idxc.at[j]], yb, sem).wait()

        @plsc.parallel_loop(0, BE, unroll=4)
        def _(e):
            cwv = _vbcast(wb[e, :], z16)
            wb[e, :] = (xb[e, :] - yb[e, :]) * cwv

        pltpu.sync_copy(mb, sh128.at[idxc.at[j]], add=True)
        pltpu.sync_copy(wb, sh16.at[idxc.at[j]], add=True)

    fire(0, bufs[0])

    def body(i, carry):
        for b in range(2):
            j = i * 2 + b

            @pl.when(j + 1 < nblk)
            def _():
                fire(j + 1, bufs[1 - b])

            @pl.when(j < nblk)
            def _():
                drain(j, bufs[b])
        return carry

    lax.fori_loop(0, (nblk + 1) // 2, body, 0)
    plsc.subcore_barrier()
    off = c * N + r0
    pltpu.sync_copy(sh128.at[pl.ds(r0, ROWS)], aggp_hbm.at[pl.ds(off, ROWS)])
    pltpu.sync_copy(sh16.at[pl.ds(r0, ROWS)], cdp_hbm.at[pl.ds(off, ROWS)])


@functools.lru_cache(maxsize=None)
def _sc_scatter(nblk, full_init):
    return pl.kernel(
        functools.partial(_sc_scatter_body, nblk, full_init),
        out_type=[
            jax.ShapeDtypeStruct((NC * N, D), jnp.float32),
            jax.ShapeDtypeStruct((NC * N, ED), jnp.float32),
        ],
        mesh=_mesh(),
        scratch_types=[
            pltpu.VMEM((nblk, BE), jnp.int32),
            pltpu.VMEM((nblk, BE), jnp.int32),
            pltpu.VMEM((BE, D), jnp.float32),
            pltpu.VMEM((BE, ED), jnp.float32),
            pltpu.VMEM((BE, ED), jnp.float32),
            pltpu.VMEM((BE, ED), jnp.float32),
            pltpu.SemaphoreType.DMA,
            pltpu.VMEM((BE, D), jnp.float32),
            pltpu.VMEM((BE, ED), jnp.float32),
            pltpu.VMEM((BE, ED), jnp.float32),
            pltpu.VMEM((BE, ED), jnp.float32),
            pltpu.SemaphoreType.DMA,
            pltpu.VMEM_SHARED((N, D), jnp.float32),
            pltpu.VMEM_SHARED((N, ED), jnp.float32),
        ],
        compiler_params=pltpu.CompilerParams(use_tc_tiling_on_sc=False),
    )


_NB_PRE = 10
_BN = N // _NB_PRE


def _tc_pre_body(h_ref, w1a, w1b, wn1a, bn1, p_ref, q_ref, r_ref):
    h = h_ref[:]
    p_ref[:] = jnp.dot(h, w1a[:], preferred_element_type=jnp.float32)
    q_ref[:] = jnp.dot(h, w1b[:], preferred_element_type=jnp.float32)
    r_ref[:] = jnp.dot(h, wn1a[:], preferred_element_type=jnp.float32) + bn1[:]


def _tc_pre(h, w1a, w1b, wn1a, bn1):
    full = lambda shp: pl.BlockSpec(shp, lambda i: (0, 0))
    blk = pl.BlockSpec((_BN, D), lambda i: (i, 0))
    return pl.pallas_call(
        _tc_pre_body,
        grid=(_NB_PRE,),
        in_specs=[blk, full((D, D)), full((D, D)), full((D, D)), full((1, D))],
        out_specs=[blk, blk, blk],
        out_shape=[jax.ShapeDtypeStruct((N, D), jnp.float32)] * 3,
    )(h, w1a, w1b, wn1a, bn1)


def _ln(t, g, b):
    mu = jnp.mean(t, axis=-1, keepdims=True)
    var = jnp.mean((t - mu) ** 2, axis=-1, keepdims=True)
    return (t - mu) / jnp.sqrt(var + 1e-5) * g + b


def _silu(t):
    return t * jax.nn.sigmoid(t)


def _tc_edge_body(g_ref, ea_ref,
                  w1e, b1, gm, btm, w2e, b2e,
                  wc1, bc1, wc2p, bc2,
                  msg_ref, cw_ref):
    t = (g_ref[:]
         + jnp.dot(ea_ref[:], w1e[:], preferred_element_type=jnp.float32)
         + b1[:])
    t = _silu(_ln(t, gm[:], btm[:]))
    m2 = jnp.dot(t, w2e[:], preferred_element_type=jnp.float32) + b2e[:]
    att = jax.nn.sigmoid(m2[:, D:D + 1])
    msg = m2[:, :D] * att
    msg_ref[:] = msg
    c1 = _silu(jnp.dot(msg, wc1[:], preferred_element_type=jnp.float32) + bc1[:])
    cwf = jnp.dot(c1, wc2p[:], preferred_element_type=jnp.float32)
    cw = jnp.clip(cwf[:, 0:1] + bc2[:], -100.0, 100.0)
    lanes = lax.broadcasted_iota(jnp.int32, (1, ED), 1)
    cw_ref[:] = jnp.where(lanes == 0, cw, 0.0)


def _tc_edge(blk_rows, blk_off, g, ea,
             w1e, b1, gm, btm, w2e, b2e, wc1, bc1, wc2p, bc2):
    esz = g.shape[0]
    full = lambda shp: pl.BlockSpec(shp, lambda i: (0, 0))
    blkD = pl.BlockSpec((blk_rows, D), lambda i: (i, 0))
    blkE = pl.BlockSpec((blk_rows, ED), lambda i: (i, 0))
    blkEoff = pl.BlockSpec((blk_rows, ED), lambda i: (i + blk_off, 0))
    return pl.pallas_call(
        _tc_edge_body,
        grid=(esz // blk_rows,),
        in_specs=[blkD, blkEoff,
                  full((ED, D)), full((1, D)), full((1, D)),
                  full((1, D)), full((D, 2 * D)), full((1, 2 * D)),
                  full((D, D)), full((1, D)),
                  full((D, D)), full((1, 1))],
        out_specs=[blkD, blkE],
        out_shape=[jax.ShapeDtypeStruct((esz, D), jnp.float32),
                   jax.ShapeDtypeStruct((esz, ED), jnp.float32)],
    )(g, ea, w1e, b1, gm, btm, w2e, b2e, wc1, bc1, wc2p, bc2)


def _tc_node_body(h_ref, r_ref, a0_ref, a1_ref, c0_ref, c1_ref, xp_ref,
                  wn1b, gm, btm, wn2, b2n,
                  hout, xout):
    t = r_ref[:] + jnp.dot(a0_ref[:] + a1_ref[:], wn1b[:],
                           preferred_element_type=jnp.float32)
    t = _silu(_ln(t, gm[:], btm[:]))
    hout[:] = h_ref[:] + jnp.dot(t, wn2[:], preferred_element_type=jnp.float32) + b2n[:]
    xout[:] = xp_ref[:] + c0_ref[:] + c1_ref[:]


def _tc_node(h, r, aggp, cdp, xp, wn1b, gm, btm, wn2, b2n):
    full = lambda shp: pl.BlockSpec(shp, lambda i: (0, 0))
    blkD = pl.BlockSpec((_BN, D), lambda i: (i, 0))
    blkE = pl.BlockSpec((_BN, ED), lambda i: (i, 0))
    blkD1 = pl.BlockSpec((_BN, D), lambda i: (i + _NB_PRE, 0))
    blkE1 = pl.BlockSpec((_BN, ED), lambda i: (i + _NB_PRE, 0))
    return pl.pallas_call(
        _tc_node_body,
        grid=(_NB_PRE,),
        in_specs=[blkD, blkD, blkD, blkD1, blkE, blkE1, blkE,
                  full((D, D)), full((1, D)), full((1, D)), full((D, D)),
                  full((1, D))],
        out_specs=[blkD, blkE],
        out_shape=[jax.ShapeDtypeStruct((N, D), jnp.float32),
                   jax.ShapeDtypeStruct((N, ED), jnp.float32)],
    )(h, r, aggp, aggp, cdp, cdp, xp, wn1b, gm, btm, wn2, b2n)


def kernel(h, x, edge_index, edge_attr,
           W_msg1, b_msg1, g_msg1, be_msg1, W_msg2, b_msg2,
           W_att, b_att,
           W_node1, b_node1, g_node1, be_node1, W_node2, b_node2,
           W_coord1, b_coord1, W_coord2, b_coord2):
    row = edge_index[0]
    col = edge_index[1]
    w1a = W_msg1[0:D]
    w1b = W_msg1[D:2 * D]
    w1d = W_msg1[2 * D:2 * D + 1]
    w1e = W_msg1[2 * D + 1:]
    wn1a = W_node1[0:D]
    wn1b = W_node1[D:]
    xpad = jnp.pad(x, ((0, 0), (0, ED - 3)))

    row2 = lambda v: v.reshape(1, -1)

    w2e = jnp.concatenate(
        [W_msg2, W_msg2 @ W_att, jnp.zeros((D, D - 1), jnp.float32)], axis=1)
    b2e = jnp.concatenate(
        [b_msg2, b_msg2 @ W_att + b_att, jnp.zeros((D - 1,), jnp.float32)]
    ).reshape(1, 2 * D)
    wc2p = jnp.concatenate(
        [W_coord2, jnp.zeros((D, D - 1), jnp.float32)], axis=1)

    rowA = row[:E_A].reshape(NW, NBLK_A, BE)
    rowB = row[E_A:].reshape(NW, NBLK_B, BE)
    colA = col[:E_A].reshape(NW, NBLK_A, BE)
    colB = col[E_A:].reshape(NW, NBLK_B, BE)
    assert E_A % 2560 == 0 and E_B % 2560 == 0

    edge_w = (w1e, row2(b_msg1), row2(g_msg1), row2(be_msg1),
              w2e, b2e, W_coord1, row2(b_coord1), wc2p,
              b_coord2.reshape(1, 1))

    p, q, r = _tc_pre(h, w1a, w1b, wn1a, row2(b_node1))
    (gA,) = _sc_gather(NBLK_A)(p, q, xpad, rowA, colA, w1d)
    (gB,) = _sc_gather(NBLK_B)(p, q, xpad, rowB, colB, w1d)
    mA, cwA = _tc_edge(2560, 0, gA, edge_attr, *edge_w)
    z128 = jnp.zeros((N, D), jnp.float32)
    z16 = jnp.zeros((N, ED), jnp.float32)
    aggA, cdA = _sc_scatter(NBLK_A, False)(mA, cwA, xpad, rowA, colA, z128, z16)
    mB, cwB = _tc_edge(2560, E_A // 2560, gB, edge_attr, *edge_w)
    aggp, cdp = _sc_scatter(NBLK_B, True)(mB, cwB, xpad, rowB, colB, aggA, cdA)
    h_new, xnp = _tc_node(
        h, r, aggp, cdp, xpad,
        wn1b, row2(g_node1), row2(be_node1), W_node2, row2(b_node2))
    return (h_new, xnp[:, :3])
